# baseline (speedup 1.0000x reference)
import functools

import jax
import jax.numpy as jnp
from jax.experimental import pallas as pl
from jax.experimental.pallas import tpu as pltpu

_NEG = float(jnp.finfo(jnp.float32).min)
_H = 128
_W = 128
_HW = _H * _W
_HWC = 2048


def _matvec_body(b_ref, w_ref, f_ref, out_ref):
    fb = f_ref[0]
    w = w_ref[...]
    out_ref[0] = jnp.dot(w, fb, preferred_element_type=jnp.float32) + b_ref[0]


def _pool_body(x_ref, out_ref):
    x = x_ref[0]
    ninf_row = jnp.full((1, _W), _NEG, jnp.float32)
    up = jnp.concatenate([x[1:], ninf_row], axis=0)
    dn = jnp.concatenate([ninf_row, x[:-1]], axis=0)
    v = jnp.maximum(jnp.maximum(x, up), dn)
    ninf_col = jnp.full((_H, 1), _NEG, jnp.float32)
    lf = jnp.concatenate([v[:, 1:], ninf_col], axis=1)
    rt = jnp.concatenate([ninf_col, v[:, :-1]], axis=1)
    p = jnp.maximum(jnp.maximum(v, lf), rt)
    out_ref[0] = jnp.where(p == x, x, _NEG)


def _masked_obj(feat, W, b):
    B, C, H, Wd = feat.shape
    fr = feat.reshape(B, C, H * Wd)
    obj = pl.pallas_call(
        _matvec_body,
        grid=(B, _HW // _HWC),
        in_specs=[
            pl.BlockSpec(memory_space=pltpu.SMEM),
            pl.BlockSpec((1, 256), lambda bi, hi: (0, 0)),
            pl.BlockSpec((1, 256, _HWC), lambda bi, hi: (bi, 0, hi)),
        ],
        out_specs=pl.BlockSpec((1, 1, _HWC), lambda bi, hi: (bi * (_HW // _HWC) + hi, 0, 0)),
        out_shape=jax.ShapeDtypeStruct((B * (_HW // _HWC), 1, _HWC), jnp.float32),
        compiler_params=pltpu.CompilerParams(
            dimension_semantics=("parallel", "arbitrary"),
        ),
    )(b, W, fr)
    masked = pl.pallas_call(
        _pool_body,
        grid=(B,),
        in_specs=[pl.BlockSpec((1, _H, _W), lambda bi: (bi, 0, 0))],
        out_specs=pl.BlockSpec((1, _H, _W), lambda bi: (bi, 0, 0)),
        out_shape=jax.ShapeDtypeStruct((B, _H, _W), jnp.float32),
        compiler_params=pltpu.CompilerParams(
            dimension_semantics=("parallel",),
        ),
    )(obj.reshape(B, _H, _W))
    return masked


def kernel(feat, W, b):
    masked = _masked_obj(feat, W, b).reshape(feat.shape[0], -1)
    sel_scores, top_inds = jax.lax.top_k(masked, 1000)
    return sel_scores, top_inds


# speedup vs baseline: 1.0177x; 1.0177x over previous
import functools

import jax
import jax.numpy as jnp
from jax.experimental import pallas as pl
from jax.experimental.pallas import tpu as pltpu

_NEG = float(jnp.finfo(jnp.float32).min)
_H = 128
_W = 128
_HW = _H * _W
_HWC = 2048


def _obj_pool_body(b_ref, w_ref, f_ref, out_ref, acc_ref):
    hi = pl.program_id(1)
    nhs = pl.num_programs(1)
    fb = f_ref[0]
    w = w_ref[...]
    part = jnp.dot(w, fb, preferred_element_type=jnp.float32) + b_ref[0]
    acc_ref[0, pl.ds(hi * _HWC, _HWC)] = part[0]

    @pl.when(hi == nhs - 1)
    def _fin():
        x = acc_ref[...].reshape(_H, _W)
        ninf_row = jnp.full((1, _W), _NEG, jnp.float32)
        up = jnp.concatenate([x[1:], ninf_row], axis=0)
        dn = jnp.concatenate([ninf_row, x[:-1]], axis=0)
        v = jnp.maximum(jnp.maximum(x, up), dn)
        ninf_col = jnp.full((_H, 1), _NEG, jnp.float32)
        lf = jnp.concatenate([v[:, 1:], ninf_col], axis=1)
        rt = jnp.concatenate([ninf_col, v[:, :-1]], axis=1)
        p = jnp.maximum(jnp.maximum(v, lf), rt)
        out_ref[0] = jnp.where(p == x, x, _NEG)


def _masked_obj(feat, W, b):
    B, C, H, Wd = feat.shape
    fr = feat.reshape(B, C, H * Wd)
    return pl.pallas_call(
        _obj_pool_body,
        grid=(B, _HW // _HWC),
        in_specs=[
            pl.BlockSpec(memory_space=pltpu.SMEM),
            pl.BlockSpec((1, C), lambda bi, hi: (0, 0)),
            pl.BlockSpec((1, C, _HWC), lambda bi, hi: (bi, 0, hi)),
        ],
        out_specs=pl.BlockSpec((1, _H, _W), lambda bi, hi: (bi, 0, 0)),
        out_shape=jax.ShapeDtypeStruct((B, _H, _W), jnp.float32),
        scratch_shapes=[pltpu.VMEM((1, _HW), jnp.float32)],
        compiler_params=pltpu.CompilerParams(
            dimension_semantics=("parallel", "arbitrary"),
        ),
    )(b, W, fr)


def kernel(feat, W, b):
    masked = _masked_obj(feat, W, b).reshape(feat.shape[0], -1)
    sel_scores, top_inds = jax.lax.top_k(masked, 1000)
    return sel_scores, top_inds


# speedup vs baseline: 1.2277x; 1.2063x over previous
import functools

import jax
import jax.numpy as jnp
from jax import lax
from jax.experimental import pallas as pl
from jax.experimental.pallas import tpu as pltpu
from jax.experimental.pallas import tpu_sc as plsc

_NEG = float(jnp.finfo(jnp.float32).min)
_NEGT = -1e38
_H = 128
_W = 128
_HW = _H * _W
_B = 8
_K = 1000

_CH = 4096
_S = 1024
_SP = _S + 16
_L = 16


def _obj_pool_body(b_ref, w_ref, f_ref, out_ref):
    fb = f_ref[0]
    w = w_ref[...]
    part = jnp.dot(w, fb, preferred_element_type=jnp.float32) + b_ref[0]
    x = part.reshape(_H, _W)
    ninf_row = jnp.full((1, _W), _NEG, jnp.float32)
    up = jnp.concatenate([x[1:], ninf_row], axis=0)
    dn = jnp.concatenate([ninf_row, x[:-1]], axis=0)
    v = jnp.maximum(jnp.maximum(x, up), dn)
    ninf_col = jnp.full((_H, 1), _NEG, jnp.float32)
    lf = jnp.concatenate([v[:, 1:], ninf_col], axis=1)
    rt = jnp.concatenate([ninf_col, v[:, :-1]], axis=1)
    p = jnp.maximum(jnp.maximum(v, lf), rt)
    out_ref[0] = jnp.where(p == x, x, _NEG)


def _masked_obj(feat, W, b):
    B, C, H, Wd = feat.shape
    fr = feat.reshape(B, C, H * Wd)
    return pl.pallas_call(
        _obj_pool_body,
        grid=(B,),
        in_specs=[
            pl.BlockSpec(memory_space=pltpu.SMEM),
            pl.BlockSpec((1, C), lambda bi: (0, 0)),
            pl.BlockSpec((1, C, _HW), lambda bi: (bi, 0, 0)),
        ],
        out_specs=pl.BlockSpec((1, _H, _W), lambda bi: (bi, 0, 0)),
        out_shape=jax.ShapeDtypeStruct((B, _H, _W), jnp.float32),
        compiler_params=pltpu.CompilerParams(
            dimension_semantics=("parallel",),
        ),
    )(b, W, fr)


def _fill_neg(ref):
    def body(i, c):
        ref[pl.ds(i * _L, _L)] = jnp.full((_L,), _NEG, jnp.float32)
        return c
    lax.fori_loop(0, _SP // _L, body, 0)


def _bitonic32(a, ai, b, bi):
    br = lax.rev(b, (0,))
    bir = lax.rev(bi, (0,))
    wa = a >= br
    hi = jnp.where(wa, a, br)
    hii = jnp.where(wa, ai, bir)
    lo = jnp.where(wa, br, a)
    loi = jnp.where(wa, bir, ai)
    hi, hii = plsc.sort_key_val(hi, hii, descending=True)
    lo, loi = plsc.sort_key_val(lo, loi, descending=True)
    return hi, hii, lo, loi


def _merge_desc(srcA_v, srcA_i, offA, nA, srcB_v, srcB_i, offB, nB,
                dst_v, dst_i, offD, T):
    a0 = srcA_v[pl.ds(offA, _L)]
    ai0 = srcA_i[pl.ds(offA, _L)]
    b0 = srcB_v[pl.ds(offB, _L)]
    bi0 = srcB_i[pl.ds(offB, _L)]

    def body(t, carry):
        a, ai, b, bi, pa, pb = carry
        hi, hii, lo, loi = _bitonic32(a, ai, b, bi)
        dst_v[pl.ds(offD + t * _L, _L)] = hi
        dst_i[pl.ds(offD + t * _L, _L)] = hii
        nxA = srcA_v[pl.ds(offA + pa * _L, _L)]
        nxAi = srcA_i[pl.ds(offA + pa * _L, _L)]
        nxB = srcB_v[pl.ds(offB + pb * _L, _L)]
        nxBi = srcB_i[pl.ds(offB + pb * _L, _L)]
        headA = nxA[0]
        headB = nxB[0]
        a_ok = jnp.where(pa < nA, 1, 0)
        b_ok = jnp.where(pb < nB, 1, 0)
        ta32 = a_ok * jnp.maximum(1 - b_ok,
                                  jnp.where(headA >= headB, 1, 0))
        ex32 = (1 - a_ok) * (1 - b_ok)
        tA = jnp.broadcast_to(ta32, (_L,)) != 0
        ex = jnp.broadcast_to(ex32, (_L,)) != 0
        b_new = jnp.where(ex, jnp.full((_L,), _NEG, jnp.float32),
                          jnp.where(tA, nxA, nxB))
        bi_new = jnp.where(tA, nxAi, nxBi)
        return lo, loi, b_new, bi_new, pa + ta32, pb + (1 - ta32)

    a, ai, b, bi, _, _ = lax.fori_loop(
        0, T - 1, body,
        (a0, ai0, b0, bi0, jnp.int32(1), jnp.int32(1)))
    hi, hii, _, _ = _bitonic32(a, ai, b, bi)
    dst_v[pl.ds(offD + (T - 1) * _L, _L)] = hi
    dst_i[pl.ds(offD + (T - 1) * _L, _L)] = hii


def _tie_fixup(vals, inds):
    for p in range(4):
        parity = p % 2

        def fix_body(s, c):
            pos = (lax.iota(jnp.int32, _L) + s * _L) * 2 + parity
            v0 = plsc.load_gather(vals, [pos])
            v1 = plsc.load_gather(vals, [pos + 1])
            i0 = plsc.load_gather(inds, [pos])
            i1 = plsc.load_gather(inds, [pos + 1])
            sw = jnp.logical_and(v0 == v1, i0 > i1)
            n0 = jnp.where(sw, i1, i0)
            n1 = jnp.where(sw, i0, i1)
            plsc.store_scatter(inds, [pos], n0)
            plsc.store_scatter(inds, [pos + 1], n1)
            return c

        lax.fori_loop(0, _S // 2 // _L, fix_body, 0)


def _topk_body(masked_hbm, out_v_hbm, out_i_hbm,
               inbuf, va, ia, vb, ib,
               pv1, pi1, pv2, pi2, pv3, pi3,
               m01v, m01i, m23v, m23i,
               sh_v, sh_i):
    cid = lax.axis_index("c")
    sid = lax.axis_index("s")
    bat = cid * 4 + sid // 4
    q = sid % 4
    base = q * _CH

    for ref in (va, vb, pv1, pv2, pv3, m01v, m23v):
        _fill_neg(ref)

    pltpu.sync_copy(masked_hbm.at[bat, pl.ds(base, _CH)], inbuf)

    def comp_body(i, cnt):
        v = inbuf[pl.ds(i * _L, _L)]
        m = v > _NEGT
        idxv = lax.iota(jnp.int32, _L) + (base + i * _L)
        off = jnp.minimum(cnt, _S - _L)
        plsc.store_compressed(va.at[pl.ds(off, _L)], v, mask=m)
        plsc.store_compressed(ia.at[pl.ds(off, _L)], idxv, mask=m)
        pc = plsc.all_reduce_population_count(m)
        return cnt + pc[0]

    lax.fori_loop(0, _CH // _L, comp_body, jnp.int32(0))

    def base_body(i, c):
        k = va[pl.ds(i * _L, _L)]
        v = ia[pl.ds(i * _L, _L)]
        k2, v2 = plsc.sort_key_val(k, v, descending=True)
        va[pl.ds(i * _L, _L)] = k2
        ia[pl.ds(i * _L, _L)] = v2
        return c
    lax.fori_loop(0, _S // _L, base_body, 0)

    src = (va, ia)
    dst = (vb, ib)
    m_v = 1
    while m_v < _S // _L:
        sv, si = src
        dv, di = dst

        def lvl_body(p, c, sv=sv, si=si, dv=dv, di=di, m_v=m_v):
            offA = p * (2 * m_v * _L)
            offB = offA + m_v * _L
            _merge_desc(sv, si, offA, m_v, sv, si, offB, m_v,
                        dv, di, offA, 2 * m_v)
            return c
        lax.fori_loop(0, _S // _L // (2 * m_v), lvl_body, 0)
        src, dst = dst, src
        m_v *= 2
    sv, si = src

    pltpu.sync_copy(sv.at[pl.ds(0, _S)], sh_v.at[sid])
    pltpu.sync_copy(si.at[pl.ds(0, _S)], sh_i.at[sid])
    plsc.subcore_barrier()

    @pl.when(q == 0)
    def _leader():
        pltpu.sync_copy(sh_v.at[sid + 1], pv1.at[pl.ds(0, _S)])
        pltpu.sync_copy(sh_i.at[sid + 1], pi1.at[pl.ds(0, _S)])
        pltpu.sync_copy(sh_v.at[sid + 2], pv2.at[pl.ds(0, _S)])
        pltpu.sync_copy(sh_i.at[sid + 2], pi2.at[pl.ds(0, _S)])
        pltpu.sync_copy(sh_v.at[sid + 3], pv3.at[pl.ds(0, _S)])
        pltpu.sync_copy(sh_i.at[sid + 3], pi3.at[pl.ds(0, _S)])
        nv = _S // _L
        _merge_desc(sv, si, 0, nv, pv1, pi1, 0, nv, m01v, m01i, 0, nv)
        _merge_desc(pv2, pi2, 0, nv, pv3, pi3, 0, nv, m23v, m23i, 0, nv)
        _merge_desc(m01v, m01i, 0, nv, m23v, m23i, 0, nv, vb, ib, 0, nv)
        _tie_fixup(vb, ib)
        pltpu.sync_copy(vb.at[pl.ds(0, _S)], out_v_hbm.at[bat])
        pltpu.sync_copy(ib.at[pl.ds(0, _S)], out_i_hbm.at[bat])


def _sc_topk(masked_flat):
    mesh = plsc.VectorSubcoreMesh(core_axis_name="c", subcore_axis_name="s",
                                  num_cores=2, num_subcores=16)
    f32 = jnp.float32
    i32 = jnp.int32
    run = pl.kernel(
        _topk_body,
        out_type=[jax.ShapeDtypeStruct((_B, _S), f32),
                  jax.ShapeDtypeStruct((_B, _S), i32)],
        mesh=mesh,
        compiler_params=pltpu.CompilerParams(needs_layout_passes=False),
        scratch_types=[
            pltpu.VMEM((_CH,), f32),
            pltpu.VMEM((_SP,), f32), pltpu.VMEM((_SP,), i32),
            pltpu.VMEM((_SP,), f32), pltpu.VMEM((_SP,), i32),
            pltpu.VMEM((_SP,), f32), pltpu.VMEM((_SP,), i32),
            pltpu.VMEM((_SP,), f32), pltpu.VMEM((_SP,), i32),
            pltpu.VMEM((_SP,), f32), pltpu.VMEM((_SP,), i32),
            pltpu.VMEM((_SP,), f32), pltpu.VMEM((_SP,), i32),
            pltpu.VMEM((_SP,), f32), pltpu.VMEM((_SP,), i32),
            pltpu.VMEM_SHARED((16, _S), f32),
            pltpu.VMEM_SHARED((16, _S), i32),
        ],
    )
    return run(masked_flat)


def kernel(feat, W, b):
    masked = _masked_obj(feat, W, b).reshape(_B, _HW)
    vals, inds = _sc_topk(masked)
    return vals[:, :_K], inds[:, :_K]
